# Initial kernel scaffold; baseline (speedup 1.0000x reference)
import jax
import jax.numpy as jnp
from jax.experimental import pallas as pl


def kernel(input, emb0, W0, emb1, W1, emb2, W2):
    raise NotImplementedError("write your pallas kernel here")


# speedup vs baseline: 1.1973x; 1.1973x over previous
import functools

import jax
import jax.numpy as jnp
from jax import lax
from jax.experimental import pallas as pl
from jax.experimental.pallas import tpu as pltpu
from jax.experimental.pallas import tpu_sc as plsc

_C0 = 20000
_C1 = 60000
_C2 = 100000
_D0, _D1, _D2 = 1024, 256, 64
_D2P = 128
_OUT = 1024

_NC, _NS, _L = 2, 16, 16
_NW = _NC * _NS
_B = 8192
_BPW = _B // _NW
_CHUNK = 64
_NCHUNK = _BPW // _CHUNK


def _sc_gather_body(ids_hbm, emb0, emb1, emb2, x0_hbm, x1_hbm, x2_hbm,
                    ids_v, i0_v, i1_v, i2_v, r0_v, r1_v, r2_v, sem):
    wid = lax.axis_index("s") * _NC + lax.axis_index("c")
    base = wid * _BPW
    pltpu.sync_copy(ids_hbm.at[pl.ds(base, _BPW)], ids_v)
    for c in range(_NCHUNK):
        for j in range(_CHUNK // _L):
            t = ids_v[pl.ds(c * _CHUNK + j * _L, _L)]
            i0_v[c, pl.ds(j * _L, _L)] = jnp.clip(t, 0, _C0 - 1)
            i1_v[c, pl.ds(j * _L, _L)] = jnp.clip(t - _C0, 0, (_C1 - _C0) - 1)
            i2_v[c, pl.ds(j * _L, _L)] = jnp.clip(t - _C1, 0, (_C2 - _C1) - 1)
    for c in range(_NCHUNK):
        cp0 = pltpu.async_copy(emb0.at[i0_v.at[c]], r0_v, sem)
        cp1 = pltpu.async_copy(emb1.at[i1_v.at[c]], r1_v, sem)
        cp2 = pltpu.async_copy(emb2.at[i2_v.at[c]], r2_v, sem)
        cp0.wait()
        cp1.wait()
        cp2.wait()
        off = base + c * _CHUNK
        pltpu.sync_copy(r0_v, x0_hbm.at[pl.ds(off, _CHUNK)])
        pltpu.sync_copy(r1_v, x1_hbm.at[pl.ds(off, _CHUNK)])
        pltpu.sync_copy(r2_v, x2_hbm.at[pl.ds(off, _CHUNK)])


_sc_gather = pl.kernel(
    _sc_gather_body,
    out_type=(
        jax.ShapeDtypeStruct((_B, _D0), jnp.float32),
        jax.ShapeDtypeStruct((_B, _D1), jnp.float32),
        jax.ShapeDtypeStruct((_B, _D2P), jnp.float32),
    ),
    mesh=plsc.VectorSubcoreMesh(core_axis_name="c", subcore_axis_name="s"),
    scratch_types=[
        pltpu.VMEM((_BPW,), jnp.int32),
        pltpu.VMEM((_NCHUNK, _CHUNK), jnp.int32),
        pltpu.VMEM((_NCHUNK, _CHUNK), jnp.int32),
        pltpu.VMEM((_NCHUNK, _CHUNK), jnp.int32),
        pltpu.VMEM((_CHUNK, _D0), jnp.float32),
        pltpu.VMEM((_CHUNK, _D1), jnp.float32),
        pltpu.VMEM((_CHUNK, _D2P), jnp.float32),
        pltpu.SemaphoreType.DMA,
    ],
)

_BT = 512


def _tc_body(ids_ref, x0_ref, x1_ref, x2_ref, w0_ref, w1_ref, w2_ref, out_ref):
    t = ids_ref[...]
    m0 = (t < _C0).astype(jnp.float32)
    m1 = jnp.logical_and(t >= _C0, t < _C1).astype(jnp.float32)
    m2 = (t >= _C1).astype(jnp.float32)
    acc = jnp.dot(x0_ref[...] * m0, w0_ref[...], preferred_element_type=jnp.float32)
    acc += jnp.dot(x1_ref[...] * m1, w1_ref[...], preferred_element_type=jnp.float32)
    acc += jnp.dot(x2_ref[...] * m2, w2_ref[...], preferred_element_type=jnp.float32)
    out_ref[...] = acc


@functools.partial(jax.jit, static_argnames=())
def _run(ids, emb0, w0t, emb1, w1t, emb2, w2t):
    x0, x1, x2 = _sc_gather(ids, emb0, emb1, emb2)
    ids2d = ids.reshape(_B, 1)
    grid = _B // _BT
    out = pl.pallas_call(
        _tc_body,
        grid=(grid,),
        in_specs=[
            pl.BlockSpec((_BT, 1), lambda i: (i, 0)),
            pl.BlockSpec((_BT, _D0), lambda i: (i, 0)),
            pl.BlockSpec((_BT, _D1), lambda i: (i, 0)),
            pl.BlockSpec((_BT, _D2P), lambda i: (i, 0)),
            pl.BlockSpec((_D0, _OUT), lambda i: (0, 0)),
            pl.BlockSpec((_D1, _OUT), lambda i: (0, 0)),
            pl.BlockSpec((_D2P, _OUT), lambda i: (0, 0)),
        ],
        out_specs=pl.BlockSpec((_BT, _OUT), lambda i: (i, 0)),
        out_shape=jax.ShapeDtypeStruct((_B, _OUT), jnp.float32),
    )(ids2d, x0, x1, x2, w0t, w1t, w2t)
    return out


def kernel(input, emb0, W0, emb1, W1, emb2, W2):
    ids = input.reshape(-1).astype(jnp.int32)
    emb2p = jnp.pad(emb2, ((0, 0), (0, _D2P - _D2)))
    w2tp = jnp.pad(W2.T, ((0, _D2P - _D2), (0, 0)))
    out = _run(ids, emb0, W0.T, emb1, W1.T, emb2p, w2tp)
    return out.reshape(input.shape + (_OUT,))


# speedup vs baseline: 1.2671x; 1.0583x over previous
import functools

import jax
import jax.numpy as jnp
from jax import lax
from jax.experimental import pallas as pl
from jax.experimental.pallas import tpu as pltpu
from jax.experimental.pallas import tpu_sc as plsc

_C0 = 20000
_C1 = 60000
_C2 = 100000
_D0, _D1, _D2 = 1024, 256, 64
_D2P = 128
_OUT = 1024

_NC, _NS, _L = 2, 16, 16
_NW = _NC * _NS
_B = 8192
_BPW = _B // _NW
_CHUNK = 32
_NCHUNK = _BPW // _CHUNK


def _sc_gather_body(ids_hbm, emb0, emb1, emb2, x0_hbm, x1_hbm, x2_hbm,
                    ids_v, i0_v, i1_v, i2_v,
                    r0a, r0b, r1a, r1b, r2a, r2b, gsem, wsem):
    wid = lax.axis_index("s") * _NC + lax.axis_index("c")
    base = wid * _BPW
    pltpu.sync_copy(ids_hbm.at[pl.ds(base, _BPW)], ids_v)
    for c in range(_NCHUNK):
        for j in range(_CHUNK // _L):
            t = ids_v[pl.ds(c * _CHUNK + j * _L, _L)]
            i0_v[c, pl.ds(j * _L, _L)] = jnp.clip(t, 0, _C0 - 1)
            i1_v[c, pl.ds(j * _L, _L)] = jnp.clip(t - _C0, 0, (_C1 - _C0) - 1)
            i2_v[c, pl.ds(j * _L, _L)] = jnp.clip(t - _C1, 0, (_C2 - _C1) - 1)
    r0 = (r0a, r0b)
    r1 = (r1a, r1b)
    r2 = (r2a, r2b)

    def fire_gather(c, b):
        return (pltpu.async_copy(emb0.at[i0_v.at[c]], r0[b], gsem),
                pltpu.async_copy(emb1.at[i1_v.at[c]], r1[b], gsem),
                pltpu.async_copy(emb2.at[i2_v.at[c]], r2[b], gsem))

    def fire_write(c, b):
        off = base + c * _CHUNK
        return (pltpu.async_copy(r0[b], x0_hbm.at[pl.ds(off, _CHUNK)], wsem),
                pltpu.async_copy(r1[b], x1_hbm.at[pl.ds(off, _CHUNK)], wsem),
                pltpu.async_copy(r2[b], x2_hbm.at[pl.ds(off, _CHUNK)], wsem))

    g = [None] * _NCHUNK
    w = [None] * _NCHUNK
    g[0] = fire_gather(0, 0)
    for c in range(_NCHUNK):
        if c + 1 < _NCHUNK:
            if c >= 1:
                for cp in w[c - 1]:
                    cp.wait()
            g[c + 1] = fire_gather(c + 1, (c + 1) % 2)
        for cp in g[c]:
            cp.wait()
        w[c] = fire_write(c, c % 2)
    for cp in w[_NCHUNK - 2]:
        cp.wait()
    for cp in w[_NCHUNK - 1]:
        cp.wait()


_sc_gather = pl.kernel(
    _sc_gather_body,
    out_type=(
        jax.ShapeDtypeStruct((_B, _D0), jnp.float32),
        jax.ShapeDtypeStruct((_B, _D1), jnp.float32),
        jax.ShapeDtypeStruct((_B, _D2P), jnp.float32),
    ),
    mesh=plsc.VectorSubcoreMesh(core_axis_name="c", subcore_axis_name="s"),
    scratch_types=[
        pltpu.VMEM((_BPW,), jnp.int32),
        pltpu.VMEM((_NCHUNK, _CHUNK), jnp.int32),
        pltpu.VMEM((_NCHUNK, _CHUNK), jnp.int32),
        pltpu.VMEM((_NCHUNK, _CHUNK), jnp.int32),
        pltpu.VMEM((_CHUNK, _D0), jnp.float32),
        pltpu.VMEM((_CHUNK, _D0), jnp.float32),
        pltpu.VMEM((_CHUNK, _D1), jnp.float32),
        pltpu.VMEM((_CHUNK, _D1), jnp.float32),
        pltpu.VMEM((_CHUNK, _D2P), jnp.float32),
        pltpu.VMEM((_CHUNK, _D2P), jnp.float32),
        pltpu.SemaphoreType.DMA,
        pltpu.SemaphoreType.DMA,
    ],
)

_BT = 512


def _tc_body(ids_ref, x0_ref, x1_ref, x2_ref, w0_ref, w1_ref, w2_ref, out_ref):
    t = ids_ref[...]
    m0 = (t < _C0).astype(jnp.float32)
    m1 = jnp.logical_and(t >= _C0, t < _C1).astype(jnp.float32)
    m2 = (t >= _C1).astype(jnp.float32)
    acc = jnp.dot(x0_ref[...] * m0, w0_ref[...], preferred_element_type=jnp.float32)
    acc += jnp.dot(x1_ref[...] * m1, w1_ref[...], preferred_element_type=jnp.float32)
    acc += jnp.dot(x2_ref[...] * m2, w2_ref[...], preferred_element_type=jnp.float32)
    out_ref[...] = acc


@functools.partial(jax.jit, static_argnames=())
def _run(ids, emb0, w0t, emb1, w1t, emb2, w2t):
    x0, x1, x2 = _sc_gather(ids, emb0, emb1, emb2)
    ids2d = ids.reshape(_B, 1)
    grid = _B // _BT
    out = pl.pallas_call(
        _tc_body,
        grid=(grid,),
        in_specs=[
            pl.BlockSpec((_BT, 1), lambda i: (i, 0)),
            pl.BlockSpec((_BT, _D0), lambda i: (i, 0)),
            pl.BlockSpec((_BT, _D1), lambda i: (i, 0)),
            pl.BlockSpec((_BT, _D2P), lambda i: (i, 0)),
            pl.BlockSpec((_D0, _OUT), lambda i: (0, 0)),
            pl.BlockSpec((_D1, _OUT), lambda i: (0, 0)),
            pl.BlockSpec((_D2P, _OUT), lambda i: (0, 0)),
        ],
        out_specs=pl.BlockSpec((_BT, _OUT), lambda i: (i, 0)),
        out_shape=jax.ShapeDtypeStruct((_B, _OUT), jnp.float32),
    )(ids2d, x0, x1, x2, w0t, w1t, w2t)
    return out


def kernel(input, emb0, W0, emb1, W1, emb2, W2):
    ids = input.reshape(-1).astype(jnp.int32)
    emb2p = jnp.pad(emb2, ((0, 0), (0, _D2P - _D2)))
    w2tp = jnp.pad(W2.T, ((0, _D2P - _D2), (0, 0)))
    out = _run(ids, emb0, W0.T, emb1, W1.T, emb2p, w2tp)
    return out.reshape(input.shape + (_OUT,))
